# baseline (speedup 1.0000x reference)
import jax
import jax.numpy as jnp
from jax.experimental import pallas as pl
from functools import partial

_EPS = 1e-5


def _density_body(rows_ref, cols_ref, out_ref, *, inv2bw2, scale):
    rows = rows_ref[0]
    cols = cols_ref[0]
    mm = jnp.dot(rows.astype(jnp.bfloat16), cols.astype(jnp.bfloat16).T,
                 preferred_element_type=jnp.float32)
    d2 = (jnp.sum(rows * rows, axis=1, keepdims=True) - 2.0 * mm
          ) + jnp.sum(cols * cols, axis=1)[None, :]
    g = jnp.exp(d2 * (-inv2bw2))
    out_ref[...] = (jnp.sum(g, axis=1) * scale)[None, None, :]


def _fused_density(xyz_cn, bw):
    B, _, N = xyz_cn.shape
    xyz8 = jnp.transpose(jnp.pad(xyz_cn, ((0, 0), (0, 5), (0, 0))), (0, 2, 1))
    TM = min(512, N)
    grid = (B, N // TM)
    scale = 1.0 / (2.5 * bw) / N
    return pl.pallas_call(
        partial(_density_body, inv2bw2=1.0 / (2.0 * bw * bw), scale=scale),
        grid=grid,
        in_specs=[
            pl.BlockSpec((1, TM, 8), lambda b, i: (b, i, 0)),
            pl.BlockSpec((1, N, 8), lambda b, i: (b, 0, 0)),
        ],
        out_specs=pl.BlockSpec((1, 1, TM), lambda b, i: (b, 0, i)),
        out_shape=jax.ShapeDtypeStruct((B, 1, N), jnp.float32),
    )(xyz8, xyz8).reshape(B, N)


def _fps_body(xyz_ref, out_ref, *, npoint):
    xyz = xyz_ref[0]
    N = xyz.shape[0]
    lane = jax.lax.broadcasted_iota(jnp.int32, (1, npoint), 1)

    def step(i, carry):
        dist, far, acc = carry
        acc = acc + far * (lane == i).astype(jnp.int32)
        centroid = xyz_ref[0, pl.ds(far, 1), :]
        d = jnp.sum((xyz - centroid) ** 2, axis=1, keepdims=True)
        dist = jnp.minimum(dist, d)
        far = jnp.argmax(dist[:, 0]).astype(jnp.int32)
        return dist, far, acc

    dist0 = jnp.full((N, 1), 1e10, jnp.float32)
    _, _, acc = jax.lax.fori_loop(
        0, npoint, step,
        (dist0, jnp.int32(0), jnp.zeros((1, npoint), jnp.int32)))
    out_ref[...] = acc[None]


def _fused_fps(xyz, npoint):
    B, N, _ = xyz.shape
    xyz8 = jnp.pad(xyz, ((0, 0), (0, 0), (0, 5)))
    return pl.pallas_call(
        partial(_fps_body, npoint=npoint),
        grid=(B,),
        in_specs=[pl.BlockSpec((1, N, 8), lambda b: (b, 0, 0))],
        out_specs=pl.BlockSpec((1, 1, npoint), lambda b: (b, 0, 0)),
        out_shape=jax.ShapeDtypeStruct((B, 1, npoint), jnp.int32),
    )(xyz8).reshape(B, npoint)


def _square_distance(src, dst):
    return (jnp.sum(src ** 2, -1)[:, :, None]
            - 2.0 * jnp.matmul(src, jnp.transpose(dst, (0, 2, 1)))
            + jnp.sum(dst ** 2, -1)[:, None, :])


def _index_points(points, idx):
    return jax.vmap(lambda p, i: p[i])(points, idx)


def _fps_scan(xyz, npoint):
    B, N, _ = xyz.shape
    def step(state, _):
        distance, farthest = state
        centroid = _index_points(xyz, farthest[:, None])
        dist = jnp.sum((xyz - centroid) ** 2, axis=-1)
        distance = jnp.minimum(distance, dist)
        new_far = jnp.argmax(distance, axis=-1).astype(jnp.int32)
        return (distance, new_far), farthest
    init = (jnp.full((B, N), 1e10, xyz.dtype), jnp.zeros((B,), jnp.int32))
    _, cent = jax.lax.scan(step, init, None, length=npoint)
    return jnp.transpose(cent, (1, 0))


def _conv2d_bn(x, layer, act):
    W, b, g, bt = layer
    y = jnp.einsum('oc,bchw->bohw', W, x) + b[None, :, None, None]
    m = jnp.mean(y, axis=(0, 2, 3), keepdims=True)
    v = jnp.var(y, axis=(0, 2, 3), keepdims=True)
    y = (y - m) / jnp.sqrt(v + _EPS) * g[None, :, None, None] + bt[None, :, None, None]
    return jax.nn.relu(y) if act == 'relu' else jax.nn.sigmoid(y)


def _bn1d(x, g, b):
    m = jnp.mean(x, axis=(0, 2), keepdims=True)
    v = jnp.var(x, axis=(0, 2), keepdims=True)
    return (x - m) / jnp.sqrt(v + _EPS) * g[None, :, None] + b[None, :, None]


def _sa_layer(xyz, points, p, npoint, nsample, bandwidth, group_all):
    B = xyz.shape[0]
    N = xyz.shape[2]
    xyz_t = jnp.transpose(xyz, (0, 2, 1))
    points_t = jnp.transpose(points, (0, 2, 1))
    if N >= 512:
        density = _fused_density(xyz, bandwidth)
    else:
        density = jnp.mean(jnp.exp(-_square_distance(xyz_t, xyz_t)
                                   / (2.0 * bandwidth * bandwidth))
                           / (2.5 * bandwidth), axis=-1)
    inv_density = 1.0 / density
    if group_all:
        new_xyz = jnp.zeros((B, 1, 3), xyz.dtype)
        grouped_xyz = xyz_t[:, None, :, :]
        new_points = jnp.concatenate([grouped_xyz, points_t[:, None, :, :]], axis=-1)
        grouped_xyz_norm = grouped_xyz
        grouped_density = inv_density.reshape(B, 1, N, 1)
        S = 1
    else:
        fps_idx = _fps_scan(xyz_t, npoint)
        new_xyz = _index_points(xyz_t, fps_idx)
        sqr = _square_distance(new_xyz, xyz_t)
        _, idx = jax.lax.top_k(-sqr, nsample)
        grouped_xyz = _index_points(xyz_t, idx)
        grouped_xyz_norm = grouped_xyz - new_xyz[:, :, None, :]
        grouped_points = _index_points(points_t, idx)
        new_points = jnp.concatenate([grouped_xyz_norm, grouped_points], axis=-1)
        grouped_density = _index_points(inv_density[:, :, None], idx)
        S = npoint
    feat = jnp.transpose(new_points, (0, 3, 2, 1))
    for layer in p['mlp']:
        feat = _conv2d_bn(feat, layer, 'relu')
    inv_max = jnp.max(grouped_density, axis=2, keepdims=True)
    ds = jnp.transpose(grouped_density / inv_max, (0, 3, 2, 1))
    nlayers = len(p['density'])
    for i, layer in enumerate(p['density']):
        ds = _conv2d_bn(ds, layer, 'sigmoid' if i == nlayers - 1 else 'relu')
    feat = feat * ds
    w = jnp.transpose(grouped_xyz_norm, (0, 3, 2, 1))
    for layer in p['weight']:
        w = _conv2d_bn(w, layer, 'relu')
    a = jnp.transpose(feat, (0, 3, 1, 2))
    wb = jnp.transpose(w, (0, 3, 2, 1))
    out = jnp.matmul(a, wb).reshape(B, S, -1)
    out = out @ p['linear_W'] + p['linear_b']
    out = jnp.transpose(out, (0, 2, 1))
    out = jax.nn.relu(_bn1d(out, p['bn_g'], p['bn_b']))
    return jnp.transpose(new_xyz, (0, 2, 1)), out


def kernel(xyz, params):
    l0_xyz = xyz[:, :3, :]
    l0_points = xyz
    l1_xyz, l1_points = _sa_layer(l0_xyz, l0_points, params['sa1'], 128, 8, 0.1, False)
    l2_xyz, l2_points = _sa_layer(l1_xyz, l1_points, params['sa2'], 64, 16, 0.2, False)
    l3_xyz, l3_points = _sa_layer(l2_xyz, l2_points, params['sa3'], 1, None, 0.4, True)
    return l3_points.reshape(xyz.shape[0], 128)
